# baseline (speedup 1.0000x reference)
import functools

import jax
import jax.numpy as jnp
from jax import lax
from jax.experimental import pallas as pl
from jax.experimental.pallas import tpu as pltpu
from jax.experimental.pallas import tpu_sc as plsc

B = 16384
D = 64

_NC = 2
_NS = 16
_NW = _NC * _NS
_BPW = B // _NW
_CHUNK = 128
_NCHUNK = _BPW // _CHUNK


def _sc_gather_body(uid_hbm, iid_hbm, gu_tab, gi_tab, mu_tab, mi_tab,
                    gu_out, gi_out, mu_out, mi_out,
                    idx_u, idx_i, bufs, sems):
    wid = lax.axis_index("s") * _NC + lax.axis_index("c")
    base = wid * _BPW
    for j in range(_NCHUNK):
        pltpu.sync_copy(uid_hbm.at[pl.ds(base + j * _CHUNK, _CHUNK)], idx_u.at[j])
        pltpu.sync_copy(iid_hbm.at[pl.ds(base + j * _CHUNK, _CHUNK)], idx_i.at[j])
    for j in range(_NCHUNK):
        row0 = base + j * _CHUNK
        cu = idx_u.at[j]
        ci = idx_i.at[j]
        c0 = pltpu.async_copy(gu_tab.at[cu], bufs.at[0], sems.at[0])
        c1 = pltpu.async_copy(gi_tab.at[ci], bufs.at[1], sems.at[1])
        c2 = pltpu.async_copy(mu_tab.at[cu], bufs.at[2], sems.at[2])
        c3 = pltpu.async_copy(mi_tab.at[ci], bufs.at[3], sems.at[3])
        c0.wait()
        pltpu.sync_copy(bufs.at[0], gu_out.at[pl.ds(row0, _CHUNK)])
        c1.wait()
        pltpu.sync_copy(bufs.at[1], gi_out.at[pl.ds(row0, _CHUNK)])
        c2.wait()
        pltpu.sync_copy(bufs.at[2], mu_out.at[pl.ds(row0, _CHUNK)])
        c3.wait()
        pltpu.sync_copy(bufs.at[3], mi_out.at[pl.ds(row0, _CHUNK)])


def _sc_gather(user_ids, item_ids, gu_tab, gi_tab, mu_tab, mi_tab):
    mesh = plsc.VectorSubcoreMesh(core_axis_name="c", subcore_axis_name="s")
    out = jax.ShapeDtypeStruct((B, D), jnp.float32)
    return pl.kernel(
        _sc_gather_body,
        out_type=(out, out, out, out),
        mesh=mesh,
        scratch_types=[
            pltpu.VMEM((_NCHUNK, _CHUNK), jnp.int32),
            pltpu.VMEM((_NCHUNK, _CHUNK), jnp.int32),
            pltpu.VMEM((4, _CHUNK, D), jnp.float32),
            pltpu.SemaphoreType.DMA((4,)),
        ],
        compiler_params=pltpu.CompilerParams(use_tc_tiling_on_sc=False),
    )(user_ids, item_ids, gu_tab, gi_tab, mu_tab, mi_tab)


_BK = 2048


def _tc_mlp_body(gu_ref, gi_ref, mu_ref, mi_ref,
                 w1_ref, b1_ref, w2_ref, b2_ref, w3_ref, b3_ref,
                 wo_ref, bo_ref, out_ref):
    f32 = jnp.float32
    gmf = gu_ref[...] * gi_ref[...]
    h = jnp.dot(mu_ref[...], w1_ref[0:D, :], preferred_element_type=f32)
    h += jnp.dot(mi_ref[...], w1_ref[D:2 * D, :], preferred_element_type=f32)
    h = jnp.maximum(h + b1_ref[...], 0.0)
    h = jnp.maximum(jnp.dot(h, w2_ref[...], preferred_element_type=f32) + b2_ref[...], 0.0)
    h = jnp.maximum(jnp.dot(h, w3_ref[...], preferred_element_type=f32) + b3_ref[...], 0.0)
    pred = jnp.dot(gmf, wo_ref[0:D, :], preferred_element_type=f32)
    pred += jnp.dot(h, wo_ref[D:D + 32, :], preferred_element_type=f32)
    out_ref[...] = pred[:, 0] + bo_ref[0]


def _tc_mlp(gu, gi, mu, mi, W1, b1, W2, b2, W3, b3, Wout, bout):
    grid = (B // _BK,)
    row_spec = pl.BlockSpec((_BK, D), lambda i: (i, 0))
    full = lambda shape: pl.BlockSpec(shape, lambda i: tuple(0 for _ in shape))
    return pl.pallas_call(
        _tc_mlp_body,
        grid=grid,
        in_specs=[
            row_spec, row_spec, row_spec, row_spec,
            full(W1.shape), full(b1.shape), full(W2.shape), full(b2.shape),
            full(W3.shape), full(b3.shape), full(Wout.shape), full(bout.shape),
        ],
        out_specs=pl.BlockSpec((_BK,), lambda i: (i,)),
        out_shape=jax.ShapeDtypeStruct((B,), jnp.float32),
    )(gu, gi, mu, mi, W1, b1, W2, b2, W3, b3, Wout, bout)


def kernel(user_ids, item_ids, gmf_user_table, gmf_item_table, mlp_user_table,
           mlp_item_table, W1, b1, W2, b2, W3, b3, Wout, bout):
    uid = user_ids.astype(jnp.int32)
    iid = item_ids.astype(jnp.int32)
    gu, gi, mu, mi = _sc_gather(uid, iid, gmf_user_table, gmf_item_table,
                                mlp_user_table, mlp_item_table)
    return _tc_mlp(gu, gi, mu, mi, W1, b1, W2, b2, W3, b3, Wout, bout)
